# Initial kernel scaffold; baseline (speedup 1.0000x reference)
import jax
import jax.numpy as jnp
from jax.experimental import pallas as pl


def kernel(x, edge_index, edge_weight, W1, b1, W2, b2):
    raise NotImplementedError("write your pallas kernel here")


# speedup vs baseline: 1.7083x; 1.7083x over previous
import jax
import jax.numpy as jnp
from jax import lax
from jax.experimental import pallas as pl
from jax.experimental.pallas import tpu as pltpu
from jax.experimental.pallas import tpu_sc as plsc

_N = 10000
_D = 128
_H = _D // 2
_E = 320000
_LANES = 16

_NC = 2
_NS = 16
_EDGES_PER_CORE = _E // _NC
_EDGES_PER_TILE = _E // (_NC * _NS)
_K = 80
_CHUNKS = _EDGES_PER_TILE // _K
_ROWS_PER_TILE = 632
_NPAD = _ROWS_PER_TILE * _NS


def _spmm_body(ha_hbm, hb_hbm, src_hbm, dst_hbm, w_hbm, out_hbm,
               idx_src, idx_dst, wts, rows, zbuf, accum, sem):
    cid = lax.axis_index("c")
    sid = lax.axis_index("s")
    row0 = sid * _ROWS_PER_TILE
    ebase = cid * _EDGES_PER_CORE + sid * _EDGES_PER_TILE

    def _zrow(r, carry):
        for j in range(_H // _LANES):
            zbuf[r, pl.ds(j * _LANES, _LANES)] = jnp.zeros((_LANES,), jnp.float32)
        return carry
    lax.fori_loop(0, _ROWS_PER_TILE, _zrow, 0)

    for half, h_hbm in enumerate((ha_hbm, hb_hbm)):
        pltpu.sync_copy(zbuf, accum.at[pl.ds(row0, _ROWS_PER_TILE)])
        plsc.subcore_barrier()

        def _chunk(c, carry):
            base = ebase + c * _K
            pltpu.sync_copy(src_hbm.at[pl.ds(base, _K)], idx_src)
            pltpu.sync_copy(dst_hbm.at[pl.ds(base, _K)], idx_dst)
            pltpu.sync_copy(w_hbm.at[pl.ds(base, _K)], wts)
            pltpu.async_copy(h_hbm.at[idx_src], rows, sem).wait()

            def _scale(g, c2):
                w16 = wts[pl.ds(g * _LANES, _LANES)]
                e0 = g * _LANES
                for i in range(_LANES):
                    wv = jnp.full((_LANES,), w16[i], jnp.float32)
                    for j in range(_H // _LANES):
                        sl = pl.ds(j * _LANES, _LANES)
                        rows[e0 + i, sl] = rows[e0 + i, sl] * wv
                return c2
            lax.fori_loop(0, _K // _LANES, _scale, 0)

            pltpu.sync_copy(rows, accum.at[idx_dst], add=True)
            return carry
        lax.fori_loop(0, _CHUNKS, _chunk, 0)

        plsc.subcore_barrier()
        out0 = (cid * 2 + half) * _NPAD + row0
        pltpu.sync_copy(accum.at[pl.ds(row0, _ROWS_PER_TILE)],
                        out_hbm.at[pl.ds(out0, _ROWS_PER_TILE)])
        plsc.subcore_barrier()


_spmm = pl.kernel(
    _spmm_body,
    out_type=jax.ShapeDtypeStruct((_NC * 2 * _NPAD, _H), jnp.float32),
    mesh=plsc.VectorSubcoreMesh(core_axis_name="c", subcore_axis_name="s"),
    compiler_params=pltpu.CompilerParams(use_tc_tiling_on_sc=False),
    scratch_types=[
        pltpu.VMEM((_K,), jnp.int32),
        pltpu.VMEM((_K,), jnp.int32),
        pltpu.VMEM((_K,), jnp.float32),
        pltpu.VMEM((_K, _H), jnp.float32),
        pltpu.VMEM((_ROWS_PER_TILE, _H), jnp.float32),
        pltpu.VMEM_SHARED((_NPAD, _H), jnp.float32),
        pltpu.SemaphoreType.DMA,
    ],
)


_BN = 1000


def _lin1_body(x_ref, wt_ref, b_ref, o_ref):
    o_ref[...] = (jnp.dot(x_ref[...], wt_ref[...],
                          preferred_element_type=jnp.float32) + b_ref[...])


def _lin2_body(p00_ref, p01_ref, p10_ref, p11_ref, wt_ref, b_ref, o_ref):
    hl = jnp.maximum(p00_ref[...] + p10_ref[...], 0.0)
    hr = jnp.maximum(p01_ref[...] + p11_ref[...], 0.0)
    acc = jnp.dot(hl, wt_ref[:_H, :], preferred_element_type=jnp.float32)
    acc += jnp.dot(hr, wt_ref[_H:, :], preferred_element_type=jnp.float32)
    o_ref[...] = acc + b_ref[...]


def _add_body(a0_ref, a1_ref, b0_ref, b1_ref, o_ref):
    o_ref[:, :_H] = a0_ref[...] + b0_ref[...]
    o_ref[:, _H:] = a1_ref[...] + b1_ref[...]


_row_spec = pl.BlockSpec((_BN, _D), lambda i: (i, 0))
_half_spec = pl.BlockSpec((_BN, _H), lambda i: (i, 0))
_mat_spec = pl.BlockSpec((_D, _D), lambda i: (0, 0))
_bias_spec = pl.BlockSpec((1, _D), lambda i: (0, 0))
_out_f32 = jax.ShapeDtypeStruct((_N, _D), jnp.float32)

_linear1 = pl.pallas_call(
    _lin1_body, grid=(_N // _BN,),
    in_specs=[_row_spec, _mat_spec, _bias_spec],
    out_specs=_row_spec, out_shape=_out_f32)

_linear2 = pl.pallas_call(
    _lin2_body, grid=(_N // _BN,),
    in_specs=[_half_spec, _half_spec, _half_spec, _half_spec,
              _mat_spec, _bias_spec],
    out_specs=_row_spec, out_shape=_out_f32)

_addk = pl.pallas_call(
    _add_body, grid=(_N // _BN,),
    in_specs=[_half_spec, _half_spec, _half_spec, _half_spec],
    out_specs=_row_spec, out_shape=_out_f32)


def _partials(p):
    return [p[k * _NPAD:k * _NPAD + _N] for k in range(_NC * 2)]


def kernel(x, edge_index, edge_weight, W1, b1, W2, b2):
    src = edge_index[0]
    dst = edge_index[1]
    w1t = W1.T
    w2t = W2.T
    b1r = b1.reshape(1, _D)
    b2r = b2.reshape(1, _D)

    h1 = _linear1(x, w1t, b1r)
    p = _partials(_spmm(h1[:, :_H], h1[:, _H:], src, dst, edge_weight))
    h2 = _linear2(p[0], p[1], p[2], p[3], w2t, b2r)
    q = _partials(_spmm(h2[:, :_H], h2[:, _H:], src, dst, edge_weight))
    return _addk(q[0], q[1], q[2], q[3])


# speedup vs baseline: 3.5360x; 2.0699x over previous
import jax
import jax.numpy as jnp
from jax import lax
from jax.experimental import pallas as pl
from jax.experimental.pallas import tpu as pltpu
from jax.experimental.pallas import tpu_sc as plsc

_N = 10000
_D = 128
_H = _D // 2
_E = 320000
_LANES = 16

_NC = 2
_NS = 16
_EDGES_PER_CORE = _E // _NC
_EDGES_PER_TILE = _E // (_NC * _NS)
_K = 80
_CHUNKS = _EDGES_PER_TILE // _K
_ROWS_PER_TILE = 632
_NPAD = _ROWS_PER_TILE * _NS


def _spmm_body(ha_hbm, hb_hbm, src_hbm, dst_hbm, w_hbm, out_hbm,
               src_all, dst_all, w_all, rows0, rows1, zbuf, accum,
               sem0, sem1):
    cid = lax.axis_index("c")
    sid = lax.axis_index("s")
    row0 = sid * _ROWS_PER_TILE
    chunk0 = cid * (_EDGES_PER_CORE // _K) + sid * _CHUNKS

    pltpu.sync_copy(src_hbm.at[pl.ds(chunk0, _CHUNKS)], src_all)
    pltpu.sync_copy(dst_hbm.at[pl.ds(chunk0, _CHUNKS)], dst_all)
    pltpu.sync_copy(w_hbm.at[pl.ds(chunk0, _CHUNKS)], w_all)

    def _zrow(r, carry):
        for j in range(_H // _LANES):
            zbuf[r, pl.ds(j * _LANES, _LANES)] = jnp.zeros((_LANES,), jnp.float32)
        return carry
    lax.fori_loop(0, _ROWS_PER_TILE, _zrow, 0)

    for half, h_hbm in enumerate((ha_hbm, hb_hbm)):
        pltpu.sync_copy(zbuf, accum.at[pl.ds(row0, _ROWS_PER_TILE)])
        plsc.subcore_barrier()

        def _gather(c, buf, sem):
            pltpu.async_copy(h_hbm.at[src_all.at[c]], buf, sem)

        def _wait(c, buf, sem):
            pltpu.make_async_copy(h_hbm.at[src_all.at[c]], buf, sem).wait()

        def _process(c, buf):
            def _scale(g, c2):
                w16 = w_all[c, pl.ds(g * _LANES, _LANES)]
                e0 = g * _LANES
                for i in range(_LANES):
                    wv = jnp.full((_LANES,), w16[i], jnp.float32)
                    for j in range(_H // _LANES):
                        sl = pl.ds(j * _LANES, _LANES)
                        buf[e0 + i, sl] = buf[e0 + i, sl] * wv
                return c2
            lax.fori_loop(0, _K // _LANES, _scale, 0)
            pltpu.sync_copy(buf, accum.at[dst_all.at[c]], add=True)

        _gather(0, rows0, sem0)

        def _pair(j, carry):
            c0 = 2 * j
            _wait(c0, rows0, sem0)
            _gather(c0 + 1, rows1, sem1)
            _process(c0, rows0)
            _wait(c0 + 1, rows1, sem1)
            _gather(c0 + 2, rows0, sem0)
            _process(c0 + 1, rows1)
            return carry
        lax.fori_loop(0, (_CHUNKS - 1) // 2, _pair, 0)
        _wait(_CHUNKS - 1, rows0, sem0)
        _process(_CHUNKS - 1, rows0)

        plsc.subcore_barrier()
        out0 = (cid * 2 + half) * _NPAD + row0
        pltpu.sync_copy(accum.at[pl.ds(row0, _ROWS_PER_TILE)],
                        out_hbm.at[pl.ds(out0, _ROWS_PER_TILE)])
        plsc.subcore_barrier()


_spmm = pl.kernel(
    _spmm_body,
    out_type=jax.ShapeDtypeStruct((_NC * 2 * _NPAD, _H), jnp.float32),
    mesh=plsc.VectorSubcoreMesh(core_axis_name="c", subcore_axis_name="s"),
    compiler_params=pltpu.CompilerParams(use_tc_tiling_on_sc=False),
    scratch_types=[
        pltpu.VMEM((_CHUNKS, _K), jnp.int32),
        pltpu.VMEM((_CHUNKS, _K), jnp.int32),
        pltpu.VMEM((_CHUNKS, _K), jnp.float32),
        pltpu.VMEM((_K, _H), jnp.float32),
        pltpu.VMEM((_K, _H), jnp.float32),
        pltpu.VMEM((_ROWS_PER_TILE, _H), jnp.float32),
        pltpu.VMEM_SHARED((_NPAD, _H), jnp.float32),
        pltpu.SemaphoreType.DMA,
        pltpu.SemaphoreType.DMA,
    ],
)


_BN = 1000


def _lin1_body(x_ref, wt_ref, b_ref, o_ref):
    o_ref[...] = (jnp.dot(x_ref[...], wt_ref[...],
                          preferred_element_type=jnp.float32) + b_ref[...])


def _lin2_body(p00_ref, p01_ref, p10_ref, p11_ref, wt_ref, b_ref, o_ref):
    hl = jnp.maximum(p00_ref[...] + p10_ref[...], 0.0)
    hr = jnp.maximum(p01_ref[...] + p11_ref[...], 0.0)
    acc = jnp.dot(hl, wt_ref[:_H, :], preferred_element_type=jnp.float32)
    acc += jnp.dot(hr, wt_ref[_H:, :], preferred_element_type=jnp.float32)
    o_ref[...] = acc + b_ref[...]


def _add_body(a0_ref, a1_ref, b0_ref, b1_ref, o_ref):
    o_ref[:, :_H] = a0_ref[...] + b0_ref[...]
    o_ref[:, _H:] = a1_ref[...] + b1_ref[...]


_row_spec = pl.BlockSpec((_BN, _D), lambda i: (i, 0))
_half_spec = pl.BlockSpec((_BN, _H), lambda i: (i, 0))
_mat_spec = pl.BlockSpec((_D, _D), lambda i: (0, 0))
_bias_spec = pl.BlockSpec((1, _D), lambda i: (0, 0))
_out_f32 = jax.ShapeDtypeStruct((_N, _D), jnp.float32)

_linear1 = pl.pallas_call(
    _lin1_body, grid=(_N // _BN,),
    in_specs=[_row_spec, _mat_spec, _bias_spec],
    out_specs=_row_spec, out_shape=_out_f32)

_linear2 = pl.pallas_call(
    _lin2_body, grid=(_N // _BN,),
    in_specs=[_half_spec, _half_spec, _half_spec, _half_spec,
              _mat_spec, _bias_spec],
    out_specs=_row_spec, out_shape=_out_f32)

_addk = pl.pallas_call(
    _add_body, grid=(_N // _BN,),
    in_specs=[_half_spec, _half_spec, _half_spec, _half_spec],
    out_specs=_row_spec, out_shape=_out_f32)


def _partials(p):
    return [p[k * _NPAD:k * _NPAD + _N] for k in range(_NC * 2)]


def kernel(x, edge_index, edge_weight, W1, b1, W2, b2):
    src = edge_index[0].reshape(_E // _K, _K)
    dst = edge_index[1].reshape(_E // _K, _K)
    edge_weight = edge_weight.reshape(_E // _K, _K)
    w1t = W1.T
    w2t = W2.T
    b1r = b1.reshape(1, _D)
    b2r = b2.reshape(1, _D)

    h1 = _linear1(x, w1t, b1r)
    p = _partials(_spmm(h1[:, :_H], h1[:, _H:], src, dst, edge_weight))
    h2 = _linear2(p[0], p[1], p[2], p[3], w2t, b2r)
    q = _partials(_spmm(h2[:, :_H], h2[:, _H:], src, dst, edge_weight))
    return _addk(q[0], q[1], q[2], q[3])
